# Initial kernel scaffold; baseline (speedup 1.0000x reference)
import jax
import jax.numpy as jnp
from jax.experimental import pallas as pl


def kernel(params, x_ssBox, x_place_frame, x_object, x_ssCylinder, times_ssBox, times_place_frame, times_object, times_ssCylinder, times_pick, times_place, batch_ssBox, batch_place_frame, batch_object, batch_ssCylinder, batch_pick, batch_place, ei_object___close_edge___ssBox, ei_ssBox___close_edge___object, ei_place_frame___close_edge___ssBox, ei_ssBox___close_edge___place_frame, ei_place_frame___close_edge___object, ei_object___close_edge___place_frame, ei_pick___time_edge___place, ei_place___time_edge___pick, ei_object___time_edge___object, ei_ssBox___time_edge___ssBox, ei_place_frame___time_edge___place_frame, ei_ssCylinder___time_edge___ssCylinder, ei_object___pick_edge___pick, ei_pick___pick_edge___object, ei_place_frame___pick_edge___pick, ei_pick___pick_edge___place_frame, ei_ssCylinder___pick_edge___pick, ei_pick___pick_edge___ssCylinder, ei_object___place_edge___place, ei_place___place_edge___object, ei_ssCylinder___place_edge___place, ei_place___place_edge___ssCylinder, ei_place_frame___place_edge___place, ei_place___place_edge___place_frame):
    raise NotImplementedError("write your pallas kernel here")


# speedup vs baseline: 4.6973x; 4.6973x over previous
import functools
import math

import jax
import jax.numpy as jnp
from jax import lax
from jax.experimental import pallas as pl
from jax.experimental.pallas import tpu as pltpu
from jax.experimental.pallas import tpu_sc as plsc

H = 128
PE_D = 4
LAYERS = 3
G = 256
N = 10000
E = 25000

NTYPES = 6
NSLOTS = 24
NPAD = 10240
NTILES = 16
RPT = NPAD // NTILES
BN2 = 512
NB2 = NPAD // BN2
CSH = 12 * NPAD // NTILES
C = 128
NCHUNK = 13
EPT = C * NCHUNK
EPAD = EPT * NTILES
BN = 400
NB = N // BN

_FEAT = {"ssBox": 4, "place_frame": 4, "object": 4, "ssCylinder": 3}
_CONS = ["pick", "place"]
_ALL = ["ssBox", "place_frame", "object", "ssCylinder", "pick", "place"]
_TID = {t: i for i, t in enumerate(_ALL)}
_ETS = [
    ("object", "close_edge", "ssBox"), ("ssBox", "close_edge", "object"),
    ("place_frame", "close_edge", "ssBox"), ("ssBox", "close_edge", "place_frame"),
    ("place_frame", "close_edge", "object"), ("object", "close_edge", "place_frame"),
    ("pick", "time_edge", "place"), ("place", "time_edge", "pick"),
    ("object", "time_edge", "object"), ("ssBox", "time_edge", "ssBox"),
    ("place_frame", "time_edge", "place_frame"), ("ssCylinder", "time_edge", "ssCylinder"),
    ("object", "pick_edge", "pick"), ("pick", "pick_edge", "object"),
    ("place_frame", "pick_edge", "pick"), ("pick", "pick_edge", "place_frame"),
    ("ssCylinder", "pick_edge", "pick"), ("pick", "pick_edge", "ssCylinder"),
    ("object", "place_edge", "place"), ("place", "place_edge", "object"),
    ("ssCylinder", "place_edge", "place"), ("place", "place_edge", "ssCylinder"),
    ("place_frame", "place_edge", "place"), ("place", "place_edge", "place_frame"),
]
_ORDER = sorted(range(NSLOTS), key=lambda i: _TID[_ETS[i][2]])
_GROUP = [[oi for oi in _ORDER if _TID[_ETS[oi][2]] == d] for d in range(NTYPES)]
_START = [sum(len(_GROUP[j]) for j in range(d)) for d in range(NTYPES)]


def _etkey(et):
    return et[0] + "___" + et[1] + "___" + et[2]


def _sc_agg(table, srcg, dstl, zeros):
    mesh = plsc.VectorSubcoreMesh(core_axis_name="c", subcore_axis_name="s")

    @functools.partial(
        pl.kernel, mesh=mesh,
        out_type=jax.ShapeDtypeStruct((NSLOTS * NPAD, H), jnp.float32),
        scratch_types=[
            pltpu.VMEM((NCHUNK, C), jnp.int32),
            pltpu.VMEM((NCHUNK, C), jnp.int32),
            pltpu.VMEM((C, H), jnp.float32),
            pltpu.VMEM((C, H), jnp.float32),
            pltpu.SemaphoreType.DMA,
            pltpu.SemaphoreType.DMA,
            pltpu.VMEM_SHARED((NPAD, H), jnp.float32),
        ],
    )
    def k(table_hbm, srcg_hbm, dstl_hbm, zeros_hbm, out_hbm,
          src_v, dst_v, buf0, buf1, sem0, sem1, accum):
        c = lax.axis_index("c")
        s = lax.axis_index("s")
        r0 = s * RPT
        bufs = (buf0, buf1)
        sems = (sem0, sem1)

        def et_body(i, carry):
            slot = c * (NSLOTS // 2) + i
            eidx = slot * NTILES + s
            pltpu.sync_copy(srcg_hbm.at[eidx], src_v)
            pltpu.sync_copy(dstl_hbm.at[eidx], dst_v)
            pltpu.sync_copy(zeros_hbm.at[pl.ds(r0, RPT)], accum.at[pl.ds(r0, RPT)])
            plsc.subcore_barrier()
            cps = [None, None]
            cps[0] = pltpu.async_copy(table_hbm.at[src_v.at[0]], buf0, sem0)
            for j in range(NCHUNK):
                b = j & 1
                cps[b].wait()
                if j + 1 < NCHUNK:
                    nb = (j + 1) & 1
                    cps[nb] = pltpu.async_copy(
                        table_hbm.at[src_v.at[j + 1]], bufs[nb], sems[nb])
                pltpu.sync_copy(bufs[b], accum.at[dst_v.at[j]], add=True)
            plsc.subcore_barrier()
            pltpu.sync_copy(accum.at[pl.ds(r0, RPT)],
                            out_hbm.at[pl.ds(slot * NPAD + r0, RPT)])
            plsc.subcore_barrier()
            return carry

        lax.fori_loop(0, NSLOTS // 2, et_body, 0)

    return k(table, srcg, dstl, zeros)


def _sc_cnt(dstl, ones_rows, zeros):
    mesh = plsc.VectorSubcoreMesh(core_axis_name="c", subcore_axis_name="s")

    @functools.partial(
        pl.kernel, mesh=mesh,
        out_type=jax.ShapeDtypeStruct((NSLOTS * NPAD, H), jnp.float32),
        scratch_types=[
            pltpu.VMEM((NCHUNK, C), jnp.int32),
            pltpu.VMEM((C, H), jnp.float32),
            pltpu.VMEM_SHARED((NPAD, H), jnp.float32),
        ],
    )
    def k(dstl_hbm, ones_hbm, zeros_hbm, out_hbm, dst_v, ones_v, accum):
        c = lax.axis_index("c")
        s = lax.axis_index("s")
        r0 = s * RPT
        pltpu.sync_copy(ones_hbm, ones_v)

        def et_body(i, carry):
            slot = c * (NSLOTS // 2) + i
            eidx = slot * NTILES + s
            pltpu.sync_copy(dstl_hbm.at[eidx], dst_v)
            pltpu.sync_copy(zeros_hbm.at[pl.ds(r0, RPT)], accum.at[pl.ds(r0, RPT)])
            plsc.subcore_barrier()
            for j in range(NCHUNK):
                pltpu.sync_copy(ones_v, accum.at[dst_v.at[j]], add=True)
            plsc.subcore_barrier()
            pltpu.sync_copy(accum.at[pl.ds(r0, RPT)],
                            out_hbm.at[pl.ds(slot * NPAD + r0, RPT)])
            plsc.subcore_barrier()
            return carry

        lax.fori_loop(0, NSLOTS // 2, et_body, 0)

    return k(dstl, ones_rows, zeros)


def _embed(x_all, w_all, b_all):
    def body(x_ref, w_ref, b_ref, o_ref):
        o_ref[0] = jnp.dot(x_ref[0], w_ref[0],
                           preferred_element_type=jnp.float32) + b_ref[0]

    return pl.pallas_call(
        body, grid=(NTYPES, NB),
        in_specs=[pl.BlockSpec((1, BN, 8), lambda d, b: (d, b, 0)),
                  pl.BlockSpec((1, 8, H), lambda d, b: (d, 0, 0)),
                  pl.BlockSpec((1, 1, H), lambda d, b: (d, 0, 0))],
        out_specs=pl.BlockSpec((1, BN, H), lambda d, b: (d, b, 0)),
        out_shape=jax.ShapeDtypeStruct((NTYPES, N, H), jnp.float32),
    )(x_all, w_all, b_all)


def _inv(cnt2d):
    rows = cnt2d.shape[0]

    def body(c_ref, o_ref):
        c = c_ref[...]
        o_ref[...] = jnp.where(c > 0.5, 1.0 / jnp.maximum(c, 1.0), 0.0)

    return pl.pallas_call(
        body, grid=(rows // 512,),
        in_specs=[pl.BlockSpec((512, H), lambda i: (i, 0))],
        out_specs=pl.BlockSpec((512, H), lambda i: (i, 0)),
        out_shape=jax.ShapeDtypeStruct((rows, H), jnp.float32),
    )(cnt2d)


def _combine(agg_d, inv_d, h_d, wl, wr, bsum, kd):
    def body(a_ref, i_ref, h_ref, wl_ref, wr_ref, b_ref, o_ref):
        acc = jnp.dot(h_ref[...], wr_ref[...],
                      preferred_element_type=jnp.float32) + b_ref[...]
        for j in range(kd):
            acc = acc + jnp.dot(a_ref[j] * i_ref[j], wl_ref[j],
                                preferred_element_type=jnp.float32)
        o_ref[...] = jnp.maximum(acc, 0.0)

    return pl.pallas_call(
        body, grid=(NB,),
        in_specs=[pl.BlockSpec((kd, BN, H), lambda b: (0, b, 0)),
                  pl.BlockSpec((kd, BN, H), lambda b: (0, b, 0)),
                  pl.BlockSpec((BN, H), lambda b: (b, 0)),
                  pl.BlockSpec((kd, H, H), lambda b: (0, 0, 0)),
                  pl.BlockSpec((H, H), lambda b: (0, 0)),
                  pl.BlockSpec((1, H), lambda b: (0, 0))],
        out_specs=pl.BlockSpec((BN, H), lambda b: (b, 0)),
        out_shape=jax.ShapeDtypeStruct((N, H), jnp.float32),
    )(agg_d, inv_d, h_d, wl, wr, bsum)


def _pool(h3, batch3):
    def body(h_ref, b_ref, s_ref, c_ref):
        b = pl.program_id(1)

        @pl.when(b == 0)
        def _():
            s_ref[...] = jnp.zeros_like(s_ref)
            c_ref[...] = jnp.zeros_like(c_ref)

        ids = b_ref[0, 0]
        oh = (ids[:, None] == lax.broadcasted_iota(jnp.int32, (BN, G), 1)
              ).astype(jnp.float32)
        s_ref[0] += lax.dot_general(oh, h_ref[0], (((0,), (0,)), ((), ())),
                                    preferred_element_type=jnp.float32)
        c_ref[0] += jnp.broadcast_to(jnp.sum(oh, axis=0)[None, :], (8, G))

    return pl.pallas_call(
        body, grid=(NTYPES, NB),
        in_specs=[pl.BlockSpec((1, BN, H), lambda d, b: (d, b, 0)),
                  pl.BlockSpec((1, 1, BN), lambda d, b: (d * NB + b, 0, 0))],
        out_specs=[pl.BlockSpec((1, G, H), lambda d, b: (d, 0, 0)),
                   pl.BlockSpec((1, 8, G), lambda d, b: (d, 0, 0))],
        out_shape=[jax.ShapeDtypeStruct((NTYPES, G, H), jnp.float32),
                   jax.ShapeDtypeStruct((NTYPES, 8, G), jnp.float32)],
    )(h3, batch3)


def _head(s, c, w1, b1, w2p, b2p):
    def body(s_ref, c_ref, w1_ref, b1_ref, w2_ref, b2_ref, o_ref):
        cnt = c_ref[:, 0, :]
        denom = jnp.maximum(cnt, 1.0)
        pooled = jnp.sum(s_ref[...] / denom[:, :, None], axis=0)
        z = jnp.maximum(pooled, 0.0)
        z1 = jnp.maximum(jnp.dot(z, w1_ref[...],
                                 preferred_element_type=jnp.float32)
                         + b1_ref[...], 0.0)
        z2 = jnp.dot(z1, w2_ref[...], preferred_element_type=jnp.float32)
        o_ref[...] = (z2[:, 0] + b2_ref[0, 0])[None, :]

    return pl.pallas_call(
        body,
        out_shape=jax.ShapeDtypeStruct((1, G), jnp.float32),
    )(s, c, w1, b1, w2p, b2p)


def _pe(times):
    pos = times.astype(jnp.float32)[:, None]
    div = jnp.exp(jnp.arange(0, PE_D, 2, dtype=jnp.float32)
                  * (-math.log(10000.0) / PE_D))
    ang = pos * div
    return jnp.concatenate([jnp.sin(ang[:, :1]), jnp.cos(ang[:, :1]),
                            jnp.sin(ang[:, 1:]), jnp.cos(ang[:, 1:])], axis=1)


def kernel(params, *a):
    xs = {t: a[i] for i, t in enumerate(_FEAT)}
    times = {t: a[4 + i] for i, t in enumerate(_ALL)}
    batches = {t: a[10 + i] for i, t in enumerate(_ALL)}
    eis = [a[16 + i] for i in range(NSLOTS)]

    x_cols, w_rows, b_rows = [], [], []
    for t in _ALL:
        pe = _pe(times[t])
        if t in _FEAT:
            feat = jnp.concatenate([xs[t], pe], axis=1)
        else:
            feat = pe
        d_in = feat.shape[1]
        feat = jnp.pad(feat, ((0, 0), (0, 8 - d_in)))
        w = jnp.pad(params["emb_W_" + t], ((0, 8 - d_in), (0, 0)))
        x_cols.append(feat)
        w_rows.append(w)
        b_rows.append(params["emb_b_" + t].reshape(1, H))
    x_all = jnp.stack(x_cols)
    w_all = jnp.stack(w_rows)
    b_all = jnp.stack(b_rows)
    h = _embed(x_all, w_all, b_all)

    pad = EPAD - E
    pad_ar = jnp.arange(pad, dtype=jnp.int32)
    srcg_l, dstl_l = [], []
    for slot, oi in enumerate(_ORDER):
        src_t, _, _ = _ETS[oi]
        ei = eis[oi].astype(jnp.int32)
        sg = jnp.concatenate([ei[0] + _TID[src_t] * N,
                              (pad_ar % N) + _TID[src_t] * N])
        dl = jnp.concatenate([ei[1], N + (pad_ar % 16)])
        srcg_l.append(sg)
        dstl_l.append(dl)
    srcg = jnp.stack(srcg_l).reshape(NSLOTS * NTILES, NCHUNK, C)
    dstl = jnp.stack(dstl_l).reshape(NSLOTS * NTILES, NCHUNK, C)
    zeros = jnp.zeros((NPAD, H), jnp.float32)
    ones_rows = jnp.ones((C, H), jnp.float32)

    cnt_wide = _sc_cnt(dstl, ones_rows, zeros)
    inv3 = _inv(cnt_wide).reshape(NSLOTS, NPAD, H)

    for l in range(LAYERS):
        aggs = _sc_agg(h.reshape(NTYPES * N, H), srcg, dstl, zeros)
        aggs = aggs.reshape(NSLOTS, NPAD, H)
        hs = []
        for d in range(NTYPES):
            s0, kd = _START[d], len(_GROUP[d])
            keys = [_etkey(_ETS[oi]) for oi in _GROUP[d]]
            wl = jnp.stack([params["sage%d_Wl_%s" % (l, k)] for k in keys])
            wr = sum(params["sage%d_Wr_%s" % (l, k)] for k in keys)
            bs = sum(params["sage%d_bl_%s" % (l, k)] for k in keys).reshape(1, H)
            hs.append(_combine(aggs[s0:s0 + kd], inv3[s0:s0 + kd],
                               h[d], wl, wr, bs, kd))
        h = jnp.stack(hs)

    batch3 = jnp.stack([batches[t].astype(jnp.int32) for t in _ALL]
                       ).reshape(NTYPES * NB, 1, BN)
    s, c = _pool(h, batch3)
    w2p = jnp.pad(params["head_W2"], ((0, 0), (0, H - 1)))
    b2p = jnp.broadcast_to(params["head_b2"].reshape(1, 1), (1, H))
    out = _head(s, c, params["head_W1"], params["head_b1"].reshape(1, 32),
                w2p, b2p)
    return out[0]


# speedup vs baseline: 6.7789x; 1.4432x over previous
import functools
import math

import jax
import jax.numpy as jnp
from jax import lax
from jax.experimental import pallas as pl
from jax.experimental.pallas import tpu as pltpu
from jax.experimental.pallas import tpu_sc as plsc

H = 128
PE_D = 4
LAYERS = 3
G = 256
N = 10000
E = 25000

NTYPES = 6
NSLOTS = 24
NPAD = 10240
NTILES = 16
RPT = NPAD // NTILES
BN2 = 512
NB2 = NPAD // BN2
CSH = 12 * NPAD // NTILES
C = 128
NCHUNK = 13
EPT = C * NCHUNK
EPAD = EPT * NTILES
BN = 400
NB = N // BN

_FEAT = {"ssBox": 4, "place_frame": 4, "object": 4, "ssCylinder": 3}
_CONS = ["pick", "place"]
_ALL = ["ssBox", "place_frame", "object", "ssCylinder", "pick", "place"]
_TID = {t: i for i, t in enumerate(_ALL)}
_ETS = [
    ("object", "close_edge", "ssBox"), ("ssBox", "close_edge", "object"),
    ("place_frame", "close_edge", "ssBox"), ("ssBox", "close_edge", "place_frame"),
    ("place_frame", "close_edge", "object"), ("object", "close_edge", "place_frame"),
    ("pick", "time_edge", "place"), ("place", "time_edge", "pick"),
    ("object", "time_edge", "object"), ("ssBox", "time_edge", "ssBox"),
    ("place_frame", "time_edge", "place_frame"), ("ssCylinder", "time_edge", "ssCylinder"),
    ("object", "pick_edge", "pick"), ("pick", "pick_edge", "object"),
    ("place_frame", "pick_edge", "pick"), ("pick", "pick_edge", "place_frame"),
    ("ssCylinder", "pick_edge", "pick"), ("pick", "pick_edge", "ssCylinder"),
    ("object", "place_edge", "place"), ("place", "place_edge", "object"),
    ("ssCylinder", "place_edge", "place"), ("place", "place_edge", "ssCylinder"),
    ("place_frame", "place_edge", "place"), ("place", "place_edge", "place_frame"),
]
_ORDER = sorted(range(NSLOTS), key=lambda i: _TID[_ETS[i][2]])
_GROUP = [[oi for oi in _ORDER if _TID[_ETS[oi][2]] == d] for d in range(NTYPES)]
_START = [sum(len(_GROUP[j]) for j in range(d)) for d in range(NTYPES)]


def _etkey(et):
    return et[0] + "___" + et[1] + "___" + et[2]


def _sc_agg(table, srcg, dstl, zeros):
    mesh = plsc.VectorSubcoreMesh(core_axis_name="c", subcore_axis_name="s")

    @functools.partial(
        pl.kernel, mesh=mesh,
        out_type=jax.ShapeDtypeStruct((NSLOTS * NPAD, H), jnp.float32),
        scratch_types=[
            pltpu.VMEM((NCHUNK, C), jnp.int32),
            pltpu.VMEM((NCHUNK, C), jnp.int32),
            pltpu.VMEM((C, H), jnp.float32),
            pltpu.VMEM((C, H), jnp.float32),
            pltpu.SemaphoreType.DMA,
            pltpu.SemaphoreType.DMA,
            pltpu.VMEM_SHARED((NPAD, H), jnp.float32),
        ],
    )
    def k(table_hbm, srcg_hbm, dstl_hbm, zeros_hbm, out_hbm,
          src_v, dst_v, buf0, buf1, sem0, sem1, accum):
        c = lax.axis_index("c")
        s = lax.axis_index("s")
        r0 = s * RPT
        bufs = (buf0, buf1)
        sems = (sem0, sem1)

        pltpu.sync_copy(zeros_hbm.at[pl.ds(r0, RPT)], accum.at[pl.ds(r0, RPT)])
        plsc.subcore_barrier()

        def et_body(i, carry):
            slot = c * (NSLOTS // 2) + i
            eidx = slot * NTILES + s
            pltpu.sync_copy(srcg_hbm.at[eidx], src_v)
            pltpu.sync_copy(dstl_hbm.at[eidx], dst_v)
            cps = [None, None]
            cps[0] = pltpu.async_copy(table_hbm.at[src_v.at[0]], buf0, sem0)
            for j in range(NCHUNK):
                b = j & 1
                cps[b].wait()
                if j + 1 < NCHUNK:
                    nb = (j + 1) & 1
                    cps[nb] = pltpu.async_copy(
                        table_hbm.at[src_v.at[j + 1]], bufs[nb], sems[nb])
                pltpu.sync_copy(bufs[b], accum.at[dst_v.at[j]], add=True)
            plsc.subcore_barrier()
            pltpu.sync_copy(accum.at[pl.ds(r0, RPT)],
                            out_hbm.at[pl.ds(slot * NPAD + r0, RPT)])
            plsc.subcore_barrier()
            return carry

        lax.fori_loop(0, NSLOTS // 2, et_body, 0)

    return k(table, srcg, dstl, zeros)


def _sc_cnt(dstl, ones_rows, zeros):
    mesh = plsc.VectorSubcoreMesh(core_axis_name="c", subcore_axis_name="s")

    @functools.partial(
        pl.kernel, mesh=mesh,
        out_type=jax.ShapeDtypeStruct((NSLOTS * NPAD, H), jnp.float32),
        scratch_types=[
            pltpu.VMEM((NCHUNK, C), jnp.int32),
            pltpu.VMEM((C, H), jnp.float32),
            pltpu.VMEM_SHARED((NPAD, H), jnp.float32),
        ],
    )
    def k(dstl_hbm, ones_hbm, zeros_hbm, out_hbm, dst_v, ones_v, accum):
        c = lax.axis_index("c")
        s = lax.axis_index("s")
        r0 = s * RPT
        pltpu.sync_copy(ones_hbm, ones_v)
        pltpu.sync_copy(zeros_hbm.at[pl.ds(r0, RPT)], accum.at[pl.ds(r0, RPT)])
        plsc.subcore_barrier()

        def et_body(i, carry):
            slot = c * (NSLOTS // 2) + i
            eidx = slot * NTILES + s
            pltpu.sync_copy(dstl_hbm.at[eidx], dst_v)
            for j in range(NCHUNK):
                pltpu.sync_copy(ones_v, accum.at[dst_v.at[j]], add=True)
            plsc.subcore_barrier()
            pltpu.sync_copy(accum.at[pl.ds(r0, RPT)],
                            out_hbm.at[pl.ds(slot * NPAD + r0, RPT)])
            plsc.subcore_barrier()
            return carry

        lax.fori_loop(0, NSLOTS // 2, et_body, 0)

    return k(dstl, ones_rows, zeros)


def _embed(x_all, w_all, b_all):
    def body(x_ref, w_ref, b_ref, o_ref):
        o_ref[0] = jnp.dot(x_ref[0], w_ref[0],
                           preferred_element_type=jnp.float32) + b_ref[0]

    return pl.pallas_call(
        body, grid=(NTYPES, NB),
        in_specs=[pl.BlockSpec((1, BN, 8), lambda d, b: (d, b, 0)),
                  pl.BlockSpec((1, 8, H), lambda d, b: (d, 0, 0)),
                  pl.BlockSpec((1, 1, H), lambda d, b: (d, 0, 0))],
        out_specs=pl.BlockSpec((1, BN, H), lambda d, b: (d, b, 0)),
        out_shape=jax.ShapeDtypeStruct((NTYPES, N, H), jnp.float32),
    )(x_all, w_all, b_all)


def _inv(cnt3):
    def body(c_ref, p_ref, o_ref):
        d = pl.program_id(0)
        first = (d % (NSLOTS // 2)) == 0
        c = jnp.where(first, c_ref[0], c_ref[0] - p_ref[0])
        o_ref[0] = jnp.where(c > 0.5, 1.0 / jnp.maximum(c, 1.0), 0.0)

    return pl.pallas_call(
        body, grid=(NSLOTS, NPAD // 512),
        in_specs=[pl.BlockSpec((1, 512, H), lambda d, b: (d, b, 0)),
                  pl.BlockSpec((1, 512, H),
                               lambda d, b: (jnp.maximum(d - 1, 0), b, 0))],
        out_specs=pl.BlockSpec((1, 512, H), lambda d, b: (d, b, 0)),
        out_shape=jax.ShapeDtypeStruct((NSLOTS, NPAD, H), jnp.float32),
    )(cnt3, cnt3)


def _combine(aggs, inv3, h_d, wl, wr, bsum, s0, kd):
    def body(*refs):
        a = refs[:kd]
        p = refs[kd:2 * kd]
        iv = refs[2 * kd:3 * kd]
        h_ref, wl_ref, wr_ref, b_ref, o_ref = refs[3 * kd:]
        acc = jnp.dot(h_ref[...], wr_ref[...],
                      preferred_element_type=jnp.float32) + b_ref[...]
        for j in range(kd):
            if (s0 + j) % (NSLOTS // 2) == 0:
                m = a[j][0]
            else:
                m = a[j][0] - p[j][0]
            acc = acc + jnp.dot(m * iv[j][0], wl_ref[j],
                                preferred_element_type=jnp.float32)
        o_ref[...] = jnp.maximum(acc, 0.0)

    agg_specs = [pl.BlockSpec((1, BN, H), lambda b, jj=s0 + j: (jj, b, 0))
                 for j in range(kd)]
    prev_specs = [pl.BlockSpec((1, BN, H),
                               lambda b, jj=max(s0 + j - 1, 0): (jj, b, 0))
                  for j in range(kd)]
    inv_specs = [pl.BlockSpec((1, BN, H), lambda b, jj=s0 + j: (jj, b, 0))
                 for j in range(kd)]
    return pl.pallas_call(
        body, grid=(NB,),
        in_specs=agg_specs + prev_specs + inv_specs + [
            pl.BlockSpec((BN, H), lambda b: (b, 0)),
            pl.BlockSpec((kd, H, H), lambda b: (0, 0, 0)),
            pl.BlockSpec((H, H), lambda b: (0, 0)),
            pl.BlockSpec((1, H), lambda b: (0, 0))],
        out_specs=pl.BlockSpec((BN, H), lambda b: (b, 0)),
        out_shape=jax.ShapeDtypeStruct((N, H), jnp.float32),
    )(*([aggs] * kd + [aggs] * kd + [inv3] * kd + [h_d, wl, wr, bsum]))


def _pool(h3, batch3):
    def body(h_ref, b_ref, s_ref, c_ref):
        b = pl.program_id(1)

        @pl.when(b == 0)
        def _():
            s_ref[...] = jnp.zeros_like(s_ref)
            c_ref[...] = jnp.zeros_like(c_ref)

        ids = b_ref[0, 0]
        oh = (ids[:, None] == lax.broadcasted_iota(jnp.int32, (BN, G), 1)
              ).astype(jnp.float32)
        s_ref[0] += lax.dot_general(oh, h_ref[0], (((0,), (0,)), ((), ())),
                                    preferred_element_type=jnp.float32)
        c_ref[0] += jnp.broadcast_to(jnp.sum(oh, axis=0)[None, :], (8, G))

    return pl.pallas_call(
        body, grid=(NTYPES, NB),
        in_specs=[pl.BlockSpec((1, BN, H), lambda d, b: (d, b, 0)),
                  pl.BlockSpec((1, 1, BN), lambda d, b: (d * NB + b, 0, 0))],
        out_specs=[pl.BlockSpec((1, G, H), lambda d, b: (d, 0, 0)),
                   pl.BlockSpec((1, 8, G), lambda d, b: (d, 0, 0))],
        out_shape=[jax.ShapeDtypeStruct((NTYPES, G, H), jnp.float32),
                   jax.ShapeDtypeStruct((NTYPES, 8, G), jnp.float32)],
    )(h3, batch3)


def _head(s, c, w1, b1, w2p, b2p):
    def body(s_ref, c_ref, w1_ref, b1_ref, w2_ref, b2_ref, o_ref):
        cnt = c_ref[:, 0, :]
        denom = jnp.maximum(cnt, 1.0)
        pooled = jnp.sum(s_ref[...] / denom[:, :, None], axis=0)
        z = jnp.maximum(pooled, 0.0)
        z1 = jnp.maximum(jnp.dot(z, w1_ref[...],
                                 preferred_element_type=jnp.float32)
                         + b1_ref[...], 0.0)
        z2 = jnp.dot(z1, w2_ref[...], preferred_element_type=jnp.float32)
        o_ref[...] = (z2[:, 0] + b2_ref[0, 0])[None, :]

    return pl.pallas_call(
        body,
        out_shape=jax.ShapeDtypeStruct((1, G), jnp.float32),
    )(s, c, w1, b1, w2p, b2p)


def _pe(times):
    pos = times.astype(jnp.float32)[:, None]
    div = jnp.exp(jnp.arange(0, PE_D, 2, dtype=jnp.float32)
                  * (-math.log(10000.0) / PE_D))
    ang = pos * div
    return jnp.concatenate([jnp.sin(ang[:, :1]), jnp.cos(ang[:, :1]),
                            jnp.sin(ang[:, 1:]), jnp.cos(ang[:, 1:])], axis=1)


def kernel(params, *a):
    xs = {t: a[i] for i, t in enumerate(_FEAT)}
    times = {t: a[4 + i] for i, t in enumerate(_ALL)}
    batches = {t: a[10 + i] for i, t in enumerate(_ALL)}
    eis = [a[16 + i] for i in range(NSLOTS)]

    x_cols, w_rows, b_rows = [], [], []
    for t in _ALL:
        pe = _pe(times[t])
        if t in _FEAT:
            feat = jnp.concatenate([xs[t], pe], axis=1)
        else:
            feat = pe
        d_in = feat.shape[1]
        feat = jnp.pad(feat, ((0, 0), (0, 8 - d_in)))
        w = jnp.pad(params["emb_W_" + t], ((0, 8 - d_in), (0, 0)))
        x_cols.append(feat)
        w_rows.append(w)
        b_rows.append(params["emb_b_" + t].reshape(1, H))
    x_all = jnp.stack(x_cols)
    w_all = jnp.stack(w_rows)
    b_all = jnp.stack(b_rows)
    h = _embed(x_all, w_all, b_all)

    pad = EPAD - E
    pad_ar = jnp.arange(pad, dtype=jnp.int32)
    srcg_l, dstl_l = [], []
    for slot, oi in enumerate(_ORDER):
        src_t, _, _ = _ETS[oi]
        ei = eis[oi].astype(jnp.int32)
        sg = jnp.concatenate([ei[0] + _TID[src_t] * N,
                              (pad_ar % N) + _TID[src_t] * N])
        dl = jnp.concatenate([ei[1], N + (pad_ar % 16)])
        srcg_l.append(sg)
        dstl_l.append(dl)
    srcg = jnp.stack(srcg_l).reshape(NSLOTS * NTILES, NCHUNK, C)
    dstl = jnp.stack(dstl_l).reshape(NSLOTS * NTILES, NCHUNK, C)
    zeros = jnp.zeros((NPAD, H), jnp.float32)
    ones_rows = jnp.ones((C, H), jnp.float32)

    cnt_wide = _sc_cnt(dstl, ones_rows, zeros)
    inv3 = _inv(cnt_wide.reshape(NSLOTS, NPAD, H))

    for l in range(LAYERS):
        aggs = _sc_agg(h.reshape(NTYPES * N, H), srcg, dstl, zeros)
        aggs = aggs.reshape(NSLOTS, NPAD, H)
        hs = []
        for d in range(NTYPES):
            s0, kd = _START[d], len(_GROUP[d])
            keys = [_etkey(_ETS[oi]) for oi in _GROUP[d]]
            wl = jnp.stack([params["sage%d_Wl_%s" % (l, k)] for k in keys])
            wr = sum(params["sage%d_Wr_%s" % (l, k)] for k in keys)
            bs = sum(params["sage%d_bl_%s" % (l, k)] for k in keys).reshape(1, H)
            hs.append(_combine(aggs, inv3, h[d], wl, wr, bs, s0, kd))
        h = jnp.stack(hs)

    batch3 = jnp.stack([batches[t].astype(jnp.int32) for t in _ALL]
                       ).reshape(NTYPES * NB, 1, BN)
    s, c = _pool(h, batch3)
    w2p = jnp.pad(params["head_W2"], ((0, 0), (0, H - 1)))
    b2p = jnp.broadcast_to(params["head_b2"].reshape(1, 1), (1, H))
    out = _head(s, c, params["head_W1"], params["head_b1"].reshape(1, 32),
                w2p, b2p)
    return out[0]
